# baseline (speedup 1.0000x reference)
import functools
import math

import jax
import jax.numpy as jnp
from jax import lax
from jax.experimental import pallas as pl
from jax.experimental.pallas import tpu as pltpu
from jax.experimental.pallas import tpu_sc as plsc

_D = 64
_B = 4096
_L = 200

_NC = 2
_NS = 16
_NW = _NC * _NS

_N = _B * _L
_ROWS_PER_W = _N // _NW
_K = 128
_CHUNKS = _ROWS_PER_W // _K
_UNROLL = 8
_SCALE = math.sqrt(_D)


def _emb_body(idx_hbm, lut_hbm, out_hbm,
              idx0, idx1, idx2, idx3, idx4,
              rows0, rows1, rows2, rows3, rows4, stg,
              isem0, isem1, isem2, isem3, isem4,
              gsem0, gsem1, gsem2, gsem3, gsem4, wsem):
    wid = lax.axis_index("s") * _NC + lax.axis_index("c")
    base = wid * _ROWS_PER_W

    idxs = (idx0, idx1, idx2, idx3, idx4)
    rows = (rows0, rows1, rows2, rows3, rows4)
    isems = (isem0, isem1, isem2, isem3, isem4)
    gsems = (gsem0, gsem1, gsem2, gsem3, gsem4)

    def fire_i(g, b):
        pltpu.make_async_copy(
            idx_hbm.at[pl.ds(base + g * _K, _K)], idxs[b], isems[b]).start()

    def wait_i(b):
        pltpu.make_async_copy(
            idx_hbm.at[pl.ds(0, _K)], idxs[b], isems[b]).wait()

    def fire_g(b):
        pltpu.make_async_copy(
            lut_hbm.at[idxs[b]], rows[b], gsems[b]).start()

    def wait_g(b):
        pltpu.make_async_copy(
            lut_hbm.at[pl.ds(0, _K), :], rows[b], gsems[b]).wait()

    def fire_w(g):
        pltpu.make_async_copy(
            stg, out_hbm.at[pl.ds(base + g * _K, _K), :], wsem).start()

    def wait_w():
        pltpu.make_async_copy(
            stg, out_hbm.at[pl.ds(0, _K), :], wsem).wait()

    def scale(rb):
        r = rows[rb]
        s = stg

        @plsc.parallel_loop(0, _K, step=1, unroll=_UNROLL)
        def body(i):
            for t in range(_D // 16):
                s[i, pl.ds(16 * t, 16)] = r[i, pl.ds(16 * t, 16)] * _SCALE

    for g in range(5):
        fire_i(g, g)
    for g in range(4):
        wait_i(g)
        fire_g(g)
    fire_w(0)

    def quint(q, carry):
        for cc in range(5):
            c = q * 5 + cc
            wait_g(cc)
            wait_w()
            scale(cc)
            fire_w(c)

            @pl.when(c + 5 < _CHUNKS)
            def _():
                fire_i(c + 5, cc)

            @pl.when(c + 4 < _CHUNKS)
            def _():
                wait_i((cc + 4) % 5)
                fire_g((cc + 4) % 5)

        return carry

    lax.fori_loop(0, _CHUNKS // 5, quint, 0)
    wait_w()


def kernel(x, lut):
    idx1d = x.reshape(_N)
    lut_pad = jnp.pad(lut, ((0, 0), (0, 128 - _D)))
    run = functools.partial(
        pl.kernel,
        mesh=plsc.VectorSubcoreMesh(core_axis_name="c", subcore_axis_name="s"),
        out_type=jax.ShapeDtypeStruct((_N, _D), jnp.float32),
        scratch_types=(
            [pltpu.VMEM((_K,), jnp.int32)] * 5
            + [pltpu.VMEM((_K, 128), jnp.float32)] * 5
            + [pltpu.VMEM((_K, _D), jnp.float32)]
            + [pltpu.SemaphoreType.DMA] * 11
        ),
        compiler_params=pltpu.CompilerParams(use_tc_tiling_on_sc=True),
    )(_emb_body)
    out = run(idx1d, lut_pad)
    return out.reshape(_B, _L, _D)


# speedup vs baseline: 1.0012x; 1.0012x over previous
import functools
import math

import jax
import jax.numpy as jnp
from jax import lax
from jax.experimental import pallas as pl
from jax.experimental.pallas import tpu as pltpu
from jax.experimental.pallas import tpu_sc as plsc

_D = 64
_B = 4096
_L = 200

_NC = 2
_NS = 16
_NW = _NC * _NS

_N = _B * _L
_ROWS_PER_W = _N // _NW
_K = 128
_CHUNKS = _ROWS_PER_W // _K
_UNROLL = 8
_SCALE = math.sqrt(_D)


def _emb_body(idx_hbm, lut_hbm, out_hbm,
              idx0, idx1, idx2, idx3, rows0, rows1, rows2, rows3,
              stg0, stg1,
              isem0, isem1, isem2, isem3,
              gsem0, gsem1, gsem2, gsem3, wsem0, wsem1):
    wid = lax.axis_index("s") * _NC + lax.axis_index("c")
    base = wid * _ROWS_PER_W

    idxs = (idx0, idx1, idx2, idx3)
    rows = (rows0, rows1, rows2, rows3)
    stgs = (stg0, stg1)
    isems = (isem0, isem1, isem2, isem3)
    gsems = (gsem0, gsem1, gsem2, gsem3)
    wsems = (wsem0, wsem1)

    def fire_i(g, b):
        pltpu.make_async_copy(
            idx_hbm.at[pl.ds(base + g * _K, _K)], idxs[b], isems[b]).start()

    def wait_i(b):
        pltpu.make_async_copy(
            idx_hbm.at[pl.ds(0, _K)], idxs[b], isems[b]).wait()

    def fire_g(b):
        pltpu.make_async_copy(
            lut_hbm.at[idxs[b]], rows[b], gsems[b]).start()

    def wait_g(b):
        pltpu.make_async_copy(
            lut_hbm.at[pl.ds(0, _K), :], rows[b], gsems[b]).wait()

    def fire_w(g, b):
        pltpu.make_async_copy(
            stgs[b], out_hbm.at[pl.ds(base + g * _K, _K), :], wsems[b]).start()

    def wait_w(b):
        pltpu.make_async_copy(
            stgs[b], out_hbm.at[pl.ds(0, _K), :], wsems[b]).wait()

    def scale(rb, sb):
        r = rows[rb]
        s = stgs[sb]

        @plsc.parallel_loop(0, _K, step=1, unroll=_UNROLL)
        def body(i):
            for t in range(_D // 16):
                s[i, pl.ds(16 * t, 16)] = r[i, pl.ds(16 * t, 16)] * _SCALE

    for g in range(4):
        fire_i(g, g)
    for g in range(3):
        wait_i(g)
        fire_g(g)
    fire_w(0, 0)
    fire_w(1, 1)

    def quad(q, carry):
        for cc in range(4):
            c = q * 4 + cc
            wait_g(cc)
            wait_w(cc % 2)
            scale(cc, cc % 2)
            fire_w(c, cc % 2)

            @pl.when(c + 4 < _CHUNKS)
            def _():
                fire_i(c + 4, cc)

            @pl.when(c + 3 < _CHUNKS)
            def _():
                wait_i((cc + 3) % 4)
                fire_g((cc + 3) % 4)

        return carry

    lax.fori_loop(0, _CHUNKS // 4, quad, 0)
    wait_w(0)
    wait_w(1)


def kernel(x, lut):
    idx1d = x.reshape(_N)
    lut_pad = jnp.pad(lut, ((0, 0), (0, 128 - _D)))
    run = functools.partial(
        pl.kernel,
        mesh=plsc.VectorSubcoreMesh(core_axis_name="c", subcore_axis_name="s"),
        out_type=jax.ShapeDtypeStruct((_N, _D), jnp.float32),
        scratch_types=(
            [pltpu.VMEM((_K,), jnp.int32)] * 4
            + [pltpu.VMEM((_K, 128), jnp.float32)] * 4
            + [pltpu.VMEM((_K, _D), jnp.float32)] * 2
            + [pltpu.SemaphoreType.DMA] * 10
        ),
        compiler_params=pltpu.CompilerParams(use_tc_tiling_on_sc=True),
    )(_emb_body)
    out = run(idx1d, lut_pad)
    return out.reshape(_B, _L, _D)
